# baseline (speedup 1.0000x reference)
import functools

import jax
import jax.numpy as jnp
from jax import lax
from jax.experimental import pallas as pl
from jax.experimental.pallas import tpu as pltpu
from jax.experimental.pallas import tpu_sc as plsc

TC = 64
LANES = 16


def _silu(v):
    return v * jax.nn.sigmoid(v)


def _meta_body(x_ref, g_ref, Ws1_ref, bs1_ref, Ws2_ref, bs2_ref, Wr_ref,
               xn_ref, base_ref, scale16_ref, dest_ref, ts_ref, tc_ref):
    N, D = x_ref.shape
    E = Wr_ref.shape[0]
    NWORK, CH = dest_ref.shape

    xv = x_ref[...]
    rs = jax.lax.rsqrt(jnp.mean(xv * xv, axis=1, keepdims=True) + 1e-6)
    xn = xv * rs * g_ref[...]
    xn_ref[...] = xn

    acc = xv
    for n in range(Ws1_ref.shape[0]):
        h = _silu(jnp.dot(xn, Ws1_ref[n], preferred_element_type=jnp.float32)
                  + bs1_ref[n])
        acc = acc + jnp.dot(h, Ws2_ref[n], preferred_element_type=jnp.float32) \
            + bs2_ref[n]
    base_ref[...] = acc

    aff = jax.lax.dot_general(xn, Wr_ref[...], (((1,), (1,)), ((), ())),
                              preferred_element_type=jnp.float32)
    idx = jnp.argmax(aff, axis=1)[:, None].astype(jnp.int32)
    scale16_ref[...] = jnp.broadcast_to(jnp.max(aff, axis=1)[:, None],
                                        scale16_ref.shape)

    eids = jax.lax.broadcasted_iota(jnp.int32, (N, E), 1)
    oh = (eids == idx).astype(jnp.float32)
    counts = jnp.sum(oh, axis=0, keepdims=True)
    tcount = jnp.floor((counts + (TC - 1)) / TC)

    er = jax.lax.broadcasted_iota(jnp.int32, (E, E), 0)
    ec = jax.lax.broadcasted_iota(jnp.int32, (E, E), 1)
    strict_lower = (er < ec).astype(jnp.float32)
    tile_start = jnp.dot(tcount, strict_lower,
                         preferred_element_type=jnp.float32)
    pof = tile_start * TC

    TB = 256
    tr = jax.lax.broadcasted_iota(jnp.int32, (TB, TB), 0)
    tcol = jax.lax.broadcasted_iota(jnp.int32, (TB, TB), 1)
    lower_inc = (tr >= tcol).astype(jnp.float32)
    carry = jnp.zeros((1, E), jnp.float32)
    for b in range(N // TB):
        blk = oh[b * TB:(b + 1) * TB]
        csum = jnp.dot(lower_inc, blk, preferred_element_type=jnp.float32) + carry
        rank = jnp.sum(blk * csum, axis=1, keepdims=True) - 1.0
        pofg = jnp.sum(blk * pof, axis=1, keepdims=True)
        dblk = (pofg + rank).astype(jnp.int32).reshape(TB // CH, CH)
        dest_ref[b * (TB // CH):(b + 1) * (TB // CH), :] = dblk
        carry = carry + jnp.sum(blk, axis=0, keepdims=True)

    ts_ref[...] = tile_start.astype(jnp.int32)
    tc_ref[...] = tcount.astype(jnp.int32)


def _gmm_body(ts_ref, tc_ref, xs_ref, W1_ref, b1_ref, W2_ref, b2_ref, y_ref):
    e = pl.program_id(0)
    row0 = ts_ref[0, e] * TC

    def tile_body(k, carry):
        off = row0 + k * TC
        h = _silu(jnp.dot(xs_ref[pl.ds(off, TC), :], W1_ref[0],
                          preferred_element_type=jnp.float32) + b1_ref[e])
        y_ref[pl.ds(off, TC), :] = jnp.dot(
            h, W2_ref[0], preferred_element_type=jnp.float32) + b2_ref[e]
        return carry

    lax.fori_loop(0, tc_ref[0, e], tile_body, 0)


def _sc_info():
    info = plsc.get_sparse_core_info()
    return info.num_cores, info.num_subcores


def _make_sc_kernels(N, D, P):
    NC, NSUB = _sc_info()
    NW = NC * NSUB
    CH = N // NW
    mesh = plsc.VectorSubcoreMesh(core_axis_name="c", subcore_axis_name="s")

    @functools.partial(
        pl.kernel, mesh=mesh,
        out_type=jax.ShapeDtypeStruct((P, D), jnp.float32),
        scratch_types=[
            pltpu.VMEM((CH,), jnp.int32),
            pltpu.VMEM((CH, D), jnp.float32),
            pltpu.SemaphoreType.DMA,
        ],
    )
    def scatter_sorted(xn_hbm, dest_hbm, xs_hbm, destv, rows, sem):
        wid = lax.axis_index("s") * NC + lax.axis_index("c")
        start = wid * CH
        pltpu.sync_copy(dest_hbm.at[wid], destv)
        pltpu.sync_copy(xn_hbm.at[pl.ds(start, CH)], rows)
        pltpu.async_copy(rows, xs_hbm.at[destv], sem).wait()

    @functools.partial(
        pl.kernel, mesh=mesh,
        out_type=jax.ShapeDtypeStruct((N, D), jnp.float32),
        scratch_types=[
            pltpu.VMEM((CH,), jnp.int32),
            pltpu.VMEM((CH, LANES), jnp.float32),
            pltpu.VMEM((CH, D), jnp.float32),
            pltpu.VMEM((CH, D), jnp.float32),
            pltpu.SemaphoreType.DMA,
        ],
    )
    def gather_combine(y_hbm, dest_hbm, scale16_hbm, base_hbm, out_hbm,
                       destv, srows, yrows, brows, sem):
        wid = lax.axis_index("s") * NC + lax.axis_index("c")
        start = wid * CH
        pltpu.sync_copy(dest_hbm.at[wid], destv)
        gath = pltpu.async_copy(y_hbm.at[destv], yrows, sem)
        pltpu.sync_copy(scale16_hbm.at[pl.ds(start, CH)], srows)
        pltpu.sync_copy(base_hbm.at[pl.ds(start, CH)], brows)
        gath.wait()

        def row_body(r, carry):
            srow = srows[r, :]
            for c in range(D // LANES):
                seg = pl.ds(c * LANES, LANES)
                brows[r, seg] = brows[r, seg] + srow * yrows[r, seg]
            return carry

        lax.fori_loop(0, CH, row_body, 0)
        pltpu.sync_copy(brows, out_hbm.at[pl.ds(start, CH)])

    return scatter_sorted, gather_combine


def kernel(x, g, Ws1, bs1, Ws2, bs2, Wr, W1, b1, W2, b2):
    B, S, D = x.shape
    NS, _, DH = Ws1.shape
    E = W1.shape[0]
    N = B * S
    NT = N // TC + E
    P = NT * TC

    xf = x.reshape(N, D)
    g2 = g.reshape(1, D)
    NC, NSUB = _sc_info()
    NW = NC * NSUB
    CH = N // NW

    xn, base, scale16, dest, ts, tcnt = pl.pallas_call(
        _meta_body,
        grid=(1,),
        in_specs=[
            pl.BlockSpec((N, D), lambda i: (0, 0)),
            pl.BlockSpec((1, D), lambda i: (0, 0)),
            pl.BlockSpec((NS, D, DH), lambda i: (0, 0, 0)),
            pl.BlockSpec((NS, 1, DH), lambda i: (0, 0, 0)),
            pl.BlockSpec((NS, DH, D), lambda i: (0, 0, 0)),
            pl.BlockSpec((NS, 1, D), lambda i: (0, 0, 0)),
            pl.BlockSpec((E, D), lambda i: (0, 0)),
        ],
        out_specs=[
            pl.BlockSpec((N, D), lambda i: (0, 0)),
            pl.BlockSpec((N, D), lambda i: (0, 0)),
            pl.BlockSpec((N, LANES), lambda i: (0, 0)),
            pl.BlockSpec((NW, CH), lambda i: (0, 0)),
            pl.BlockSpec((1, E), lambda i: (0, 0)),
            pl.BlockSpec((1, E), lambda i: (0, 0)),
        ],
        out_shape=[
            jax.ShapeDtypeStruct((N, D), jnp.float32),
            jax.ShapeDtypeStruct((N, D), jnp.float32),
            jax.ShapeDtypeStruct((N, LANES), jnp.float32),
            jax.ShapeDtypeStruct((NW, CH), jnp.int32),
            jax.ShapeDtypeStruct((1, E), jnp.int32),
            jax.ShapeDtypeStruct((1, E), jnp.int32),
        ],
    )(xf, g2, Ws1, bs1, Ws2, bs2, Wr)

    scatter_sorted, gather_combine = _make_sc_kernels(N, D, P)
    xs = scatter_sorted(xn, dest)

    grid_spec = pltpu.PrefetchScalarGridSpec(
        num_scalar_prefetch=2,
        grid=(E,),
        in_specs=[
            pl.BlockSpec((P, D), lambda e, ts_, tc_: (0, 0)),
            pl.BlockSpec((1, D, DH), lambda e, ts_, tc_: (e, 0, 0)),
            pl.BlockSpec((E, 1, DH), lambda e, ts_, tc_: (0, 0, 0)),
            pl.BlockSpec((1, DH, D), lambda e, ts_, tc_: (e, 0, 0)),
            pl.BlockSpec((E, 1, D), lambda e, ts_, tc_: (0, 0, 0)),
        ],
        out_specs=pl.BlockSpec((P, D), lambda e, ts_, tc_: (0, 0)),
    )
    y = pl.pallas_call(
        _gmm_body,
        grid_spec=grid_spec,
        out_shape=jax.ShapeDtypeStruct((P, D), jnp.float32),
    )(ts, tcnt, xs, W1, b1, W2, b2)

    out = gather_combine(y, dest, scale16, base)
    return out.reshape(B, S, D)


# speedup vs baseline: 1.0113x; 1.0113x over previous
import functools

import jax
import jax.numpy as jnp
from jax import lax
from jax.experimental import pallas as pl
from jax.experimental.pallas import tpu as pltpu
from jax.experimental.pallas import tpu_sc as plsc

TC = 64
LANES = 16


def _silu(v):
    return v * jax.nn.sigmoid(v)


def _meta_body(x_ref, g_ref, Ws1_ref, bs1_ref, Ws2_ref, bs2_ref, Wr_ref,
               xn_ref, base_ref, scale16_ref, dest_ref, te_ref, nt_ref):
    N, D = x_ref.shape
    E = Wr_ref.shape[0]
    NT = te_ref.shape[0]
    NWORK, CH = dest_ref.shape

    xv = x_ref[...]
    rs = jax.lax.rsqrt(jnp.mean(xv * xv, axis=1, keepdims=True) + 1e-6)
    xn = xv * rs * g_ref[...]
    xn_ref[...] = xn

    acc = xv
    for n in range(Ws1_ref.shape[0]):
        h = _silu(jnp.dot(xn, Ws1_ref[n], preferred_element_type=jnp.float32)
                  + bs1_ref[n])
        acc = acc + jnp.dot(h, Ws2_ref[n], preferred_element_type=jnp.float32) \
            + bs2_ref[n]
    base_ref[...] = acc

    aff = jax.lax.dot_general(xn, Wr_ref[...], (((1,), (1,)), ((), ())),
                              preferred_element_type=jnp.float32)
    idx = jnp.argmax(aff, axis=1)[:, None].astype(jnp.int32)
    scale16_ref[...] = jnp.broadcast_to(jnp.max(aff, axis=1)[:, None],
                                        scale16_ref.shape)

    eids = jax.lax.broadcasted_iota(jnp.int32, (N, E), 1)
    oh = (eids == idx).astype(jnp.float32)
    counts = jnp.sum(oh, axis=0, keepdims=True)
    tcount = jnp.floor((counts + (TC - 1)) / TC)

    er = jax.lax.broadcasted_iota(jnp.int32, (E, E), 0)
    ec = jax.lax.broadcasted_iota(jnp.int32, (E, E), 1)
    strict_lower = (er < ec).astype(jnp.float32)
    tile_start = jnp.dot(tcount, strict_lower,
                         preferred_element_type=jnp.float32)
    pof = tile_start * TC

    TB = 256
    tr = jax.lax.broadcasted_iota(jnp.int32, (TB, TB), 0)
    tcol = jax.lax.broadcasted_iota(jnp.int32, (TB, TB), 1)
    lower_inc = (tr >= tcol).astype(jnp.float32)
    carry = jnp.zeros((1, E), jnp.float32)
    for b in range(N // TB):
        blk = oh[b * TB:(b + 1) * TB]
        csum = jnp.dot(lower_inc, blk, preferred_element_type=jnp.float32) + carry
        rank = jnp.sum(blk * csum, axis=1, keepdims=True) - 1.0
        pofg = jnp.sum(blk * pof, axis=1, keepdims=True)
        dblk = (pofg + rank).astype(jnp.int32).reshape(TB // CH, CH)
        dest_ref[b * (TB // CH):(b + 1) * (TB // CH), :] = dblk
        carry = carry + jnp.sum(blk, axis=0, keepdims=True)

    tt = jax.lax.broadcasted_iota(jnp.int32, (NT, E), 0).astype(jnp.float32)
    ge = (tt >= tile_start).astype(jnp.float32)
    te_ref[...] = (jnp.sum(ge, axis=1, keepdims=True) - 1.0).astype(jnp.int32)
    nt_ref[...] = jnp.sum(tcount, axis=1, keepdims=True).astype(jnp.int32)


def _gmm_body(te_ref, nt_ref, xs_ref, W1_ref, b1_ref, W2_ref, b2_ref, y_ref):
    t = pl.program_id(0)

    @pl.when(t < nt_ref[0, 0])
    def _():
        e = te_ref[t, 0]
        h = _silu(jnp.dot(xs_ref[...], W1_ref[0],
                          preferred_element_type=jnp.float32) + b1_ref[e])
        y_ref[...] = jnp.dot(h, W2_ref[0],
                             preferred_element_type=jnp.float32) + b2_ref[e]


def _sc_info():
    info = plsc.get_sparse_core_info()
    return info.num_cores, info.num_subcores


def _make_sc_kernels(N, D, P):
    NC, NSUB = _sc_info()
    NW = NC * NSUB
    CH = N // NW
    mesh = plsc.VectorSubcoreMesh(core_axis_name="c", subcore_axis_name="s")

    @functools.partial(
        pl.kernel, mesh=mesh,
        out_type=jax.ShapeDtypeStruct((P, D), jnp.float32),
        scratch_types=[
            pltpu.VMEM((CH,), jnp.int32),
            pltpu.VMEM((CH, D), jnp.float32),
            pltpu.SemaphoreType.DMA,
        ],
    )
    def scatter_sorted(xn_hbm, dest_hbm, xs_hbm, destv, rows, sem):
        wid = lax.axis_index("s") * NC + lax.axis_index("c")
        start = wid * CH
        pltpu.sync_copy(dest_hbm.at[wid], destv)
        pltpu.sync_copy(xn_hbm.at[pl.ds(start, CH)], rows)
        pltpu.async_copy(rows, xs_hbm.at[destv], sem).wait()

    @functools.partial(
        pl.kernel, mesh=mesh,
        out_type=jax.ShapeDtypeStruct((N, D), jnp.float32),
        scratch_types=[
            pltpu.VMEM((CH,), jnp.int32),
            pltpu.VMEM((CH, LANES), jnp.float32),
            pltpu.VMEM((CH, D), jnp.float32),
            pltpu.VMEM((CH, D), jnp.float32),
            pltpu.SemaphoreType.DMA,
        ],
    )
    def gather_combine(y_hbm, dest_hbm, scale16_hbm, base_hbm, out_hbm,
                       destv, srows, yrows, brows, sem):
        wid = lax.axis_index("s") * NC + lax.axis_index("c")
        start = wid * CH
        pltpu.sync_copy(dest_hbm.at[wid], destv)
        gath = pltpu.async_copy(y_hbm.at[destv], yrows, sem)
        pltpu.sync_copy(scale16_hbm.at[pl.ds(start, CH)], srows)
        pltpu.sync_copy(base_hbm.at[pl.ds(start, CH)], brows)
        gath.wait()

        def row_body(r, carry):
            srow = srows[r, :]
            for c in range(D // LANES):
                seg = pl.ds(c * LANES, LANES)
                brows[r, seg] = brows[r, seg] + srow * yrows[r, seg]
            return carry

        lax.fori_loop(0, CH, row_body, 0)
        pltpu.sync_copy(brows, out_hbm.at[pl.ds(start, CH)])

    return scatter_sorted, gather_combine


def kernel(x, g, Ws1, bs1, Ws2, bs2, Wr, W1, b1, W2, b2):
    B, S, D = x.shape
    NS, _, DH = Ws1.shape
    E = W1.shape[0]
    N = B * S
    NT = N // TC + E
    P = NT * TC

    xf = x.reshape(N, D)
    g2 = g.reshape(1, D)
    NC, NSUB = _sc_info()
    NW = NC * NSUB
    CH = N // NW

    xn, base, scale16, dest, te, nt = pl.pallas_call(
        _meta_body,
        grid=(1,),
        in_specs=[
            pl.BlockSpec((N, D), lambda i: (0, 0)),
            pl.BlockSpec((1, D), lambda i: (0, 0)),
            pl.BlockSpec((NS, D, DH), lambda i: (0, 0, 0)),
            pl.BlockSpec((NS, 1, DH), lambda i: (0, 0, 0)),
            pl.BlockSpec((NS, DH, D), lambda i: (0, 0, 0)),
            pl.BlockSpec((NS, 1, D), lambda i: (0, 0, 0)),
            pl.BlockSpec((E, D), lambda i: (0, 0)),
        ],
        out_specs=[
            pl.BlockSpec((N, D), lambda i: (0, 0)),
            pl.BlockSpec((N, D), lambda i: (0, 0)),
            pl.BlockSpec((N, LANES), lambda i: (0, 0)),
            pl.BlockSpec((NW, CH), lambda i: (0, 0)),
            pl.BlockSpec((NT, 1), lambda i: (0, 0)),
            pl.BlockSpec((1, 1), lambda i: (0, 0)),
        ],
        out_shape=[
            jax.ShapeDtypeStruct((N, D), jnp.float32),
            jax.ShapeDtypeStruct((N, D), jnp.float32),
            jax.ShapeDtypeStruct((N, LANES), jnp.float32),
            jax.ShapeDtypeStruct((NW, CH), jnp.int32),
            jax.ShapeDtypeStruct((NT, 1), jnp.int32),
            jax.ShapeDtypeStruct((1, 1), jnp.int32),
        ],
    )(xf, g2, Ws1, bs1, Ws2, bs2, Wr)

    scatter_sorted, gather_combine = _make_sc_kernels(N, D, P)
    xs = scatter_sorted(xn, dest)

    grid_spec = pltpu.PrefetchScalarGridSpec(
        num_scalar_prefetch=2,
        grid=(NT,),
        in_specs=[
            pl.BlockSpec((TC, D),
                         lambda t, te_, nt_: (jnp.minimum(t, nt_[0, 0] - 1), 0)),
            pl.BlockSpec((1, D, DH), lambda t, te_, nt_: (te_[t, 0], 0, 0)),
            pl.BlockSpec((E, 1, DH), lambda t, te_, nt_: (0, 0, 0)),
            pl.BlockSpec((1, DH, D), lambda t, te_, nt_: (te_[t, 0], 0, 0)),
            pl.BlockSpec((E, 1, D), lambda t, te_, nt_: (0, 0, 0)),
        ],
        out_specs=pl.BlockSpec(
            (TC, D), lambda t, te_, nt_: (jnp.minimum(t, nt_[0, 0] - 1), 0)),
    )
    y = pl.pallas_call(
        _gmm_body,
        grid_spec=grid_spec,
        out_shape=jax.ShapeDtypeStruct((P, D), jnp.float32),
    )(te, nt, xs, W1, b1, W2, b2)

    out = gather_combine(y, dest, scale16, base)
    return out.reshape(B, S, D)
